# baseline (speedup 1.0000x reference)
import functools

import jax
import jax.numpy as jnp
from jax.experimental import pallas as pl
from jax.experimental.pallas import tpu as pltpu

N = 16384
E = 100000
Q = 100000
REL_SRC = (0, 1, 0, 2, 1, 2)
REL_DST = (1, 0, 2, 0, 2, 1)

QBLK = 1000


def _mlp_body(z_ref, p1w_ref, p1b_ref, p2w_ref, p2b_ref, p3w_ref, p3b_ref, o_ref):
    z = z_ref[...]
    z = jax.nn.relu(
        jax.lax.dot_general(z, p1w_ref[...], (((1,), (0,)), ((), ())),
                            precision=None)
        + p1b_ref[...])
    z = jax.nn.relu(
        jax.lax.dot_general(z, p2w_ref[...], (((1,), (0,)), ((), ())),
                            precision=None)
        + p2b_ref[...])
    o_ref[...] = (
        jax.lax.dot_general(z, p3w_ref[...], (((1,), (0,)), ((), ())),
                            precision=None)
        + p3b_ref[...])


def _predictor(z, P1W, P1b, P2W, P2b, P3W, P3b):
    grid = (Q // QBLK,)
    return pl.pallas_call(
        _mlp_body,
        grid=grid,
        in_specs=[
            pl.BlockSpec((QBLK, 256), lambda i: (i, 0)),
            pl.BlockSpec((256, 128), lambda i: (0, 0)),
            pl.BlockSpec((128,), lambda i: (0,)),
            pl.BlockSpec((128, 64), lambda i: (0, 0)),
            pl.BlockSpec((64,), lambda i: (0,)),
            pl.BlockSpec((64, 1), lambda i: (0, 0)),
            pl.BlockSpec((1,), lambda i: (0,)),
        ],
        out_specs=pl.BlockSpec((QBLK, 1), lambda i: (i, 0)),
        out_shape=jax.ShapeDtypeStruct((Q, 1), jnp.float32),
    )(z, P1W, P1b, P2W, P2b, P3W, P3b)


def _graph_conv(h_src, src, dst, W, b):
    deg_out = jnp.bincount(src, length=N).clip(1).astype(jnp.float32)
    deg_in = jnp.bincount(dst, length=N).clip(1).astype(jnp.float32)
    feat = (h_src * jax.lax.rsqrt(deg_out)[:, None]) @ W
    agg = jax.ops.segment_sum(feat[src], dst, num_segments=N)
    return agg * jax.lax.rsqrt(deg_in)[:, None] + b


def kernel(x_drug, x_protein, drug_id, dis_id, edges, Wd, bd, Wp, bp, W1, b1, W2, b2, P1W, P1b, P2W, P2b, P3W, P3b):
    h = {0: x_drug @ Wd + bd, 1: x_protein @ Wp + bp}
    out1 = {}
    for r in range(6):
        s, d = REL_SRC[r], REL_DST[r]
        if s not in h:
            continue
        m = _graph_conv(h[s], edges[r, 0], edges[r, 1], W1[r], b1[r])
        out1[d] = out1[d] + m if d in out1 else m
    h1 = {k: jax.nn.relu(v) for k, v in out1.items()}
    out2 = {}
    for r in range(6):
        s, d = REL_SRC[r], REL_DST[r]
        if s not in h1:
            continue
        m = _graph_conv(h1[s], edges[r, 0], edges[r, 1], W2[r], b2[r])
        out2[d] = out2[d] + m if d in out2 else m
    z = jnp.concatenate([out2[0][drug_id], out2[2][dis_id]], axis=-1)
    return _predictor(z, P1W, P1b, P2W, P2b, P3W, P3b)


# speedup vs baseline: 1.1440x; 1.1440x over previous
import functools

import jax
import jax.numpy as jnp
from jax import lax
from jax.experimental import pallas as pl
from jax.experimental.pallas import tpu as pltpu
from jax.experimental.pallas import tpu_sc as plsc

N = 16384
E = 100000
Q = 100000
REL_SRC = (0, 1, 0, 2, 1, 2)
REL_DST = (1, 0, 2, 0, 2, 1)

HALF = N // 2
NTILES = 16
EPT = 6256
EPAD = NTILES * EPT
CHUNK = 2048
CAP = 8192

_i32 = jnp.int32
_f32 = jnp.float32


def _bincount_body(nrows, idx_hbm, out_hbm, acc, dstbuf, didx, ones, zeros):
    c = lax.axis_index("c")
    s = lax.axis_index("s")

    def _fill(ref, n, val):
        def step(i, _):
            ref[pl.ds(i * 16, 16)] = val
            return 0
        lax.fori_loop(0, n // 16, step, 0)

    _fill(ones, CHUNK, jnp.full((16,), 1.0, _f32))
    _fill(zeros, 512, jnp.zeros((16,), _f32))
    dummy = jnp.full((16,), HALF, _i32) + lax.iota(_i32, 16)
    base = c * HALF
    def _senttail(i, _):
        dstbuf[pl.ds(EPT + i * 16, 16)] = jnp.full((16,), N, _i32)
        return 0
    lax.fori_loop(0, (CAP - EPT) // 16, _senttail, 0)

    def relbody(rr, _):
        pltpu.sync_copy(zeros, acc.at[pl.ds(s * 512, 512)])

        @pl.when(s == 0)
        def _():
            pltpu.sync_copy(zeros.at[pl.ds(0, 32)], acc.at[pl.ds(HALF, 32)])

        plsc.subcore_barrier()
        pltpu.sync_copy(idx_hbm.at[pl.ds(rr * EPAD + s * EPT, EPT)],
                        dstbuf.at[pl.ds(0, EPT)])

        gpc = CHUNK // 16
        for j in range(CAP // CHUNK):
            def xform(i, _):
                g = j * gpc + i
                vd = dstbuf[pl.ds(g * 16, 16)]
                loc = vd - base
                m = (loc >= 0) & (loc < HALF)
                didx[pl.ds(i * 16, 16)] = jnp.where(m, loc, dummy)
                return 0

            lax.fori_loop(0, gpc, xform, 0)
            pltpu.sync_copy(ones, acc.at[didx], add=True)
        plsc.subcore_barrier()
        pltpu.sync_copy(acc.at[pl.ds(s * 512, 512)],
                        out_hbm.at[pl.ds(rr * N + base + s * 512, 512)])
        plsc.subcore_barrier()
        return 0

    lax.fori_loop(0, nrows, relbody, 0)


def _sc_bincount(idx_pad, nrows):
    mesh = plsc.VectorSubcoreMesh(core_axis_name="c", subcore_axis_name="s")
    kern = functools.partial(
        pl.kernel,
        out_type=jax.ShapeDtypeStruct((nrows * N,), _f32),
        mesh=mesh,
        scratch_types=[
            pltpu.VMEM_SHARED((HALF + 32,), _f32),
            pltpu.VMEM((CAP,), _i32),
            pltpu.VMEM((CHUNK,), _i32),
            pltpu.VMEM((CHUNK,), _f32),
            pltpu.VMEM((512,), _f32),
        ],
    )(functools.partial(_bincount_body, nrows))
    return kern(idx_pad)


CHUNKR = 512
NCHR = EPT // CHUNKR + 1
CAPR = NCHR * CHUNKR
QR = N // 4
PAD = 16


def _segsum_body(nrel, npass, ar, feats_hbm, eidx_hbm, out_hbm, acc, srcbuf,
                 dstbuf, gidx, didx, rows, zrows):
    c = lax.axis_index("c")
    s = lax.axis_index("s")
    rpt = ar // 16

    def _z(i, _):
        zrows[i // 8, pl.ds((i % 8) * 16, 16)] = jnp.zeros((16,), _f32)
        return 0

    lax.fori_loop(0, 64 * 8, _z, 0)

    def _tails(i, _):
        srcbuf[pl.ds(EPT + i * 16, 16)] = lax.iota(_i32, 16)
        dstbuf[pl.ds(EPT + i * 16, 16)] = jnp.full((16,), N, _i32)
        return 0

    lax.fori_loop(0, (CAPR - EPT) // 16, _tails, 0)

    def relbody(rr, _):
        rbase = rr * N
        pltpu.sync_copy(eidx_hbm.at[pl.ds((2 * rr) * EPAD + s * EPT, EPT)],
                        srcbuf.at[pl.ds(0, EPT)])
        pltpu.sync_copy(eidx_hbm.at[pl.ds((2 * rr + 1) * EPAD + s * EPT, EPT)],
                        dstbuf.at[pl.ds(0, EPT)])
        for p in range(npass):
            base = (c * npass + p) * ar

            def _zero(k, _2):
                pltpu.sync_copy(zrows, acc.at[pl.ds(s * rpt + k * 64, 64)])
                return 0

            lax.fori_loop(0, rpt // 64, _zero, 0)

            @pl.when(s == 0)
            def _():
                pltpu.sync_copy(zrows.at[pl.ds(0, PAD)], acc.at[pl.ds(ar, PAD)])

            plsc.subcore_barrier()
            gpc = CHUNKR // 16
            for j in range(NCHR):
                dummy = jnp.full((16,), ar, _i32) + lax.iota(_i32, 16)

                def xform(i, _2):
                    g = j * gpc + i
                    vd = dstbuf[pl.ds(g * 16, 16)]
                    loc = vd - base
                    m = (loc >= 0) & (loc < ar)
                    didx[pl.ds(i * 16, 16)] = jnp.where(m, loc, dummy)
                    gidx[pl.ds(i * 16, 16)] = srcbuf[pl.ds(g * 16, 16)] + rbase
                    return 0

                lax.fori_loop(0, gpc, xform, 0)
                pltpu.sync_copy(feats_hbm.at[gidx], rows)
                pltpu.sync_copy(rows, acc.at[didx], add=True)
            plsc.subcore_barrier()
            pltpu.sync_copy(acc.at[pl.ds(s * rpt, rpt)],
                            out_hbm.at[pl.ds(rbase + base + s * rpt, rpt)])
            plsc.subcore_barrier()
        return 0

    lax.fori_loop(0, nrel, relbody, 0)


def _sc_segsum(feats, eidx, nrel, npass):
    ar = N // (2 * npass)
    mesh = plsc.VectorSubcoreMesh(core_axis_name="c", subcore_axis_name="s")
    kern = functools.partial(
        pl.kernel,
        out_type=jax.ShapeDtypeStruct((nrel * N, 128), _f32),
        mesh=mesh,
        scratch_types=[
            pltpu.VMEM_SHARED((ar + PAD, 128), _f32),
            pltpu.VMEM((CAPR,), _i32),
            pltpu.VMEM((CAPR,), _i32),
            pltpu.VMEM((CHUNKR,), _i32),
            pltpu.VMEM((CHUNKR,), _i32),
            pltpu.VMEM((CHUNKR, 128), _f32),
            pltpu.VMEM((64, 128), _f32),
        ],
    )(functools.partial(_segsum_body, nrel, npass, ar))
    return kern(feats, eidx)


QBLK = 1000


def _mlp_body(a_ref, b_ref, p2w_ref, p2b_ref, p3w_ref, p3b_ref, o_ref):
    z = jax.nn.relu(a_ref[...] + b_ref[...])
    z = jax.nn.relu(
        jax.lax.dot_general(z, p2w_ref[...], (((1,), (0,)), ((), ())))
        + p2b_ref[...])
    o_ref[...] = (
        jax.lax.dot_general(z, p3w_ref[...], (((1,), (0,)), ((), ())))
        + p3b_ref[...])


def _predictor_mlp(ag, bg, P2W, P2b, P3W, P3b):
    grid = (Q // QBLK,)
    return pl.pallas_call(
        _mlp_body,
        grid=grid,
        in_specs=[
            pl.BlockSpec((QBLK, 128), lambda i: (i, 0)),
            pl.BlockSpec((QBLK, 128), lambda i: (i, 0)),
            pl.BlockSpec((128, 64), lambda i: (0, 0)),
            pl.BlockSpec((64,), lambda i: (0,)),
            pl.BlockSpec((64, 1), lambda i: (0, 0)),
            pl.BlockSpec((1,), lambda i: (0,)),
        ],
        out_specs=pl.BlockSpec((QBLK, 1), lambda i: (i, 0)),
        out_shape=jax.ShapeDtypeStruct((Q, 1), jnp.float32),
    )(ag, bg, P2W, P2b, P3W, P3b)


def _matmul_body(x_ref, w_ref, b_ref, o_ref):
    o_ref[...] = (
        jax.lax.dot_general(x_ref[...], w_ref[...], (((1,), (0,)), ((), ())))
        + b_ref[...])


def _matmul_bias(x, w, b, blk=512):
    n, k = x.shape
    m = w.shape[1]
    return pl.pallas_call(
        _matmul_body,
        grid=(n // blk,),
        in_specs=[
            pl.BlockSpec((blk, k), lambda i: (i, 0)),
            pl.BlockSpec((k, m), lambda i: (0, 0)),
            pl.BlockSpec((m,), lambda i: (0,)),
        ],
        out_specs=pl.BlockSpec((blk, m), lambda i: (i, 0)),
        out_shape=jax.ShapeDtypeStruct((n, m), jnp.float32),
    )(x, w, b)


def kernel(x_drug, x_protein, drug_id, dis_id, edges, Wd, bd, Wp, bp, W1, b1, W2, b2, P1W, P1b, P2W, P2b, P3W, P3b):
    idx12 = edges.reshape(12, E)
    idx_pad = jnp.concatenate(
        [idx12, jnp.full((12, EPAD - E), N, _i32)], axis=1).reshape(-1)
    counts = _sc_bincount(idx_pad, 12).reshape(12, N)
    scale = jax.lax.rsqrt(jnp.maximum(counts, 1.0))

    h = {0: _matmul_bias(x_drug, Wd, bd), 1: _matmul_bias(x_protein, Wp, bp)}

    pad_src = jnp.arange(EPAD - E, dtype=_i32) % 16
    pad_dst = jnp.full((EPAD - E,), N, _i32)

    def conv(h_in, rels, W, b, npass):
        feats = jnp.concatenate(
            [_matmul_bias(h_in[REL_SRC[r]] * scale[2 * r][:, None], W[r],
                          jnp.zeros((128,), _f32)) for r in rels], axis=0)
        eidx = jnp.concatenate(
            [jnp.concatenate([edges[r, 0], pad_src, edges[r, 1], pad_dst])
             for r in rels])
        agg = _sc_segsum(feats, eidx, len(rels), npass)
        out = {}
        for k, r in enumerate(rels):
            d = REL_DST[r]
            m = (agg[k * N:(k + 1) * N] * scale[2 * r + 1][:, None] + b[r])
            out[d] = out[d] + m if d in out else m
        return out

    out1 = conv(h, (0, 1, 2, 4), W1, b1, npass=2)
    h1 = {k: jax.nn.relu(v) for k, v in out1.items()}
    out2 = conv(h1, (1, 2, 3, 4), W2, b2, npass=2)

    a = _matmul_bias(out2[0], P1W[:128], P1b)
    bz = _matmul_bias(out2[2], P1W[128:], jnp.zeros((128,), _f32))
    ag = a[drug_id]
    bg = bz[dis_id]
    return _predictor_mlp(ag, bg, P2W, P2b, P3W, P3b)


# speedup vs baseline: 1.4951x; 1.3069x over previous
import functools

import jax
import jax.numpy as jnp
from jax import lax
from jax.experimental import pallas as pl
from jax.experimental.pallas import tpu as pltpu
from jax.experimental.pallas import tpu_sc as plsc

N = 16384
E = 100000
Q = 100000
REL_SRC = (0, 1, 0, 2, 1, 2)
REL_DST = (1, 0, 2, 0, 2, 1)

HALF = N // 2
NTILES = 16
EPT = 6256
EPAD = NTILES * EPT
CHUNK = 2048
CAP = 8192

_i32 = jnp.int32
_f32 = jnp.float32


def _bincount_body(nrows, idx_hbm, out_hbm, acc, dstbuf, didx, ones, zeros):
    c = lax.axis_index("c")
    s = lax.axis_index("s")

    def _fill(ref, n, val):
        def step(i, _):
            ref[pl.ds(i * 16, 16)] = val
            return 0
        lax.fori_loop(0, n // 16, step, 0)

    _fill(ones, CHUNK, jnp.full((16,), 1.0, _f32))
    _fill(zeros, 512, jnp.zeros((16,), _f32))
    dummy = jnp.full((16,), HALF, _i32) + lax.iota(_i32, 16)
    base = c * HALF
    def _senttail(i, _):
        dstbuf[pl.ds(EPT + i * 16, 16)] = jnp.full((16,), N, _i32)
        return 0
    lax.fori_loop(0, (CAP - EPT) // 16, _senttail, 0)

    def relbody(rr, _):
        pltpu.sync_copy(zeros, acc.at[pl.ds(s * 512, 512)])

        @pl.when(s == 0)
        def _():
            pltpu.sync_copy(zeros.at[pl.ds(0, 32)], acc.at[pl.ds(HALF, 32)])

        plsc.subcore_barrier()
        pltpu.sync_copy(idx_hbm.at[pl.ds(rr * EPAD + s * EPT, EPT)],
                        dstbuf.at[pl.ds(0, EPT)])

        gpc = CHUNK // 16
        for j in range(CAP // CHUNK):
            def xform(i, _):
                g = j * gpc + i
                vd = dstbuf[pl.ds(g * 16, 16)]
                loc = vd - base
                m = (loc >= 0) & (loc < HALF)
                didx[pl.ds(i * 16, 16)] = jnp.where(m, loc, dummy)
                return 0

            lax.fori_loop(0, gpc, xform, 0)
            pltpu.sync_copy(ones, acc.at[didx], add=True)
        plsc.subcore_barrier()
        pltpu.sync_copy(acc.at[pl.ds(s * 512, 512)],
                        out_hbm.at[pl.ds(rr * N + base + s * 512, 512)])
        plsc.subcore_barrier()
        return 0

    lax.fori_loop(0, nrows, relbody, 0)


def _sc_bincount(idx_pad, nrows):
    mesh = plsc.VectorSubcoreMesh(core_axis_name="c", subcore_axis_name="s")
    kern = functools.partial(
        pl.kernel,
        out_type=jax.ShapeDtypeStruct((nrows * N,), _f32),
        mesh=mesh,
        scratch_types=[
            pltpu.VMEM_SHARED((HALF + 32,), _f32),
            pltpu.VMEM((CAP,), _i32),
            pltpu.VMEM((CHUNK,), _i32),
            pltpu.VMEM((CHUNK,), _f32),
            pltpu.VMEM((512,), _f32),
        ],
    )(functools.partial(_bincount_body, nrows))
    return kern(idx_pad)


CHUNKR = 256
NCHR = -(-EPT // CHUNKR)
CAPR = NCHR * CHUNKR
PAD = 16


def _segsum_body(nrel, npass, ar, feats_hbm, eidx_hbm, out_hbm, acc, srcbuf,
                 dstbuf, gidx0, gidx1, didx0, didx1, rows0, rows1, zrows,
                 gsem0, gsem1, ssem0, ssem1):
    c = lax.axis_index("c")
    s = lax.axis_index("s")
    rpt = ar // 16
    gidx = (gidx0, gidx1)
    didx = (didx0, didx1)
    rows = (rows0, rows1)
    gsem = (gsem0, gsem1)
    ssem = (ssem0, ssem1)

    def _z(i, _):
        zrows[i // 8, pl.ds((i % 8) * 16, 16)] = jnp.zeros((16,), _f32)
        return 0

    lax.fori_loop(0, 32 * 8, _z, 0)

    def _tails(i, _):
        srcbuf[pl.ds(EPT + i * 16, 16)] = lax.iota(_i32, 16)
        dstbuf[pl.ds(EPT + i * 16, 16)] = jnp.full((16,), N, _i32)
        return 0

    lax.fori_loop(0, (CAPR - EPT) // 16, _tails, 0)
    gpc = CHUNKR // 16

    def relbody(rr, _):
        rbase = rr * N
        pltpu.sync_copy(eidx_hbm.at[pl.ds((2 * rr) * EPAD + s * EPT, EPT)],
                        srcbuf.at[pl.ds(0, EPT)])
        pltpu.sync_copy(eidx_hbm.at[pl.ds((2 * rr + 1) * EPAD + s * EPT, EPT)],
                        dstbuf.at[pl.ds(0, EPT)])
        for p in range(npass):
            base = (c * npass + p) * ar

            def _zero(k, _2):
                pltpu.sync_copy(zrows, acc.at[pl.ds(s * rpt + k * 32, 32)])
                return 0

            lax.fori_loop(0, rpt // 32, _zero, 0)

            @pl.when(s == 0)
            def _():
                pltpu.sync_copy(zrows.at[pl.ds(0, PAD)], acc.at[pl.ds(ar, PAD)])

            plsc.subcore_barrier()
            dummy = jnp.full((16,), ar, _i32) + lax.iota(_i32, 16)

            def xform(j, b):
                def step(i, _2):
                    g = j * gpc + i
                    vd = dstbuf[pl.ds(g * 16, 16)]
                    loc = vd - base
                    m = (loc >= 0) & (loc < ar)
                    didx[b][pl.ds(i * 16, 16)] = jnp.where(m, loc, dummy)
                    gidx[b][pl.ds(i * 16, 16)] = srcbuf[pl.ds(g * 16, 16)] + rbase
                    return 0

                lax.fori_loop(0, gpc, step, 0)

            gd = [None, None]
            sd = [None, None]
            for j in range(NCHR + 1):
                b = j % 2
                if j < NCHR:
                    if j >= 2:
                        sd[b].wait()
                    xform(j, b)
                    gd[b] = pltpu.async_copy(feats_hbm.at[gidx[b]], rows[b],
                                             gsem[b])
                if j >= 1:
                    bb = (j - 1) % 2
                    gd[bb].wait()
                    sd[bb] = pltpu.async_copy(rows[bb], acc.at[didx[bb]],
                                              ssem[bb], add=True)
            sd[(NCHR - 1) % 2].wait()
            sd[NCHR % 2].wait()
            plsc.subcore_barrier()
            pltpu.sync_copy(acc.at[pl.ds(s * rpt, rpt)],
                            out_hbm.at[pl.ds(rbase + base + s * rpt, rpt)])
            plsc.subcore_barrier()
        return 0

    lax.fori_loop(0, nrel, relbody, 0)


def _sc_segsum(feats, eidx, nrel, npass):
    ar = N // (2 * npass)
    mesh = plsc.VectorSubcoreMesh(core_axis_name="c", subcore_axis_name="s")
    kern = functools.partial(
        pl.kernel,
        out_type=jax.ShapeDtypeStruct((nrel * N, 128), _f32),
        mesh=mesh,
        scratch_types=[
            pltpu.VMEM_SHARED((ar + PAD, 128), _f32),
            pltpu.VMEM((CAPR,), _i32),
            pltpu.VMEM((CAPR,), _i32),
            pltpu.VMEM((CHUNKR,), _i32),
            pltpu.VMEM((CHUNKR,), _i32),
            pltpu.VMEM((CHUNKR,), _i32),
            pltpu.VMEM((CHUNKR,), _i32),
            pltpu.VMEM((CHUNKR, 128), _f32),
            pltpu.VMEM((CHUNKR, 128), _f32),
            pltpu.VMEM((32, 128), _f32),
            pltpu.SemaphoreType.DMA,
            pltpu.SemaphoreType.DMA,
            pltpu.SemaphoreType.DMA,
            pltpu.SemaphoreType.DMA,
        ],
    )(functools.partial(_segsum_body, nrel, npass, ar))
    return kern(feats, eidx)


QBLK = 1000


def _mlp_body(a_ref, b_ref, p2w_ref, p2b_ref, p3w_ref, p3b_ref, o_ref):
    z = jax.nn.relu(a_ref[...] + b_ref[...])
    z = jax.nn.relu(
        jax.lax.dot_general(z, p2w_ref[...], (((1,), (0,)), ((), ())))
        + p2b_ref[...])
    o_ref[...] = (
        jax.lax.dot_general(z, p3w_ref[...], (((1,), (0,)), ((), ())))
        + p3b_ref[...])


def _predictor_mlp(ag, bg, P2W, P2b, P3W, P3b):
    grid = (Q // QBLK,)
    return pl.pallas_call(
        _mlp_body,
        grid=grid,
        in_specs=[
            pl.BlockSpec((QBLK, 128), lambda i: (i, 0)),
            pl.BlockSpec((QBLK, 128), lambda i: (i, 0)),
            pl.BlockSpec((128, 64), lambda i: (0, 0)),
            pl.BlockSpec((64,), lambda i: (0,)),
            pl.BlockSpec((64, 1), lambda i: (0, 0)),
            pl.BlockSpec((1,), lambda i: (0,)),
        ],
        out_specs=pl.BlockSpec((QBLK, 1), lambda i: (i, 0)),
        out_shape=jax.ShapeDtypeStruct((Q, 1), jnp.float32),
    )(ag, bg, P2W, P2b, P3W, P3b)


def _matmul_body(x_ref, w_ref, b_ref, o_ref):
    o_ref[...] = (
        jax.lax.dot_general(x_ref[...], w_ref[...], (((1,), (0,)), ((), ())))
        + b_ref[...])


def _matmul_bias(x, w, b, blk=512):
    n, k = x.shape
    m = w.shape[1]
    return pl.pallas_call(
        _matmul_body,
        grid=(n // blk,),
        in_specs=[
            pl.BlockSpec((blk, k), lambda i: (i, 0)),
            pl.BlockSpec((k, m), lambda i: (0, 0)),
            pl.BlockSpec((m,), lambda i: (0,)),
        ],
        out_specs=pl.BlockSpec((blk, m), lambda i: (i, 0)),
        out_shape=jax.ShapeDtypeStruct((n, m), jnp.float32),
    )(x, w, b)


def kernel(x_drug, x_protein, drug_id, dis_id, edges, Wd, bd, Wp, bp, W1, b1, W2, b2, P1W, P1b, P2W, P2b, P3W, P3b):
    idx12 = edges.reshape(12, E)
    idx_pad = jnp.concatenate(
        [idx12, jnp.full((12, EPAD - E), N, _i32)], axis=1).reshape(-1)
    counts = _sc_bincount(idx_pad, 12).reshape(12, N)
    scale = jax.lax.rsqrt(jnp.maximum(counts, 1.0))

    h = {0: _matmul_bias(x_drug, Wd, bd), 1: _matmul_bias(x_protein, Wp, bp)}

    pad_src = jnp.arange(EPAD - E, dtype=_i32) % 16
    pad_dst = jnp.full((EPAD - E,), N, _i32)

    def conv(h_in, rels, W, b, npass):
        feats = jnp.concatenate(
            [_matmul_bias(h_in[REL_SRC[r]] * scale[2 * r][:, None], W[r],
                          jnp.zeros((128,), _f32)) for r in rels], axis=0)
        eidx = jnp.concatenate(
            [jnp.concatenate([edges[r, 0], pad_src, edges[r, 1], pad_dst])
             for r in rels])
        agg = _sc_segsum(feats, eidx, len(rels), npass)
        out = {}
        for k, r in enumerate(rels):
            d = REL_DST[r]
            m = (agg[k * N:(k + 1) * N] * scale[2 * r + 1][:, None] + b[r])
            out[d] = out[d] + m if d in out else m
        return out

    out1 = conv(h, (0, 1, 2, 4), W1, b1, npass=2)
    h1 = {k: jax.nn.relu(v) for k, v in out1.items()}
    out2 = conv(h1, (1, 2, 3, 4), W2, b2, npass=2)

    a = _matmul_bias(out2[0], P1W[:128], P1b)
    bz = _matmul_bias(out2[2], P1W[128:], jnp.zeros((128,), _f32))
    ag = a[drug_id]
    bg = bz[dis_id]
    return _predictor_mlp(ag, bg, P2W, P2b, P3W, P3b)


# speedup vs baseline: 1.5092x; 1.0094x over previous
import functools

import jax
import jax.numpy as jnp
from jax import lax
from jax.experimental import pallas as pl
from jax.experimental.pallas import tpu as pltpu
from jax.experimental.pallas import tpu_sc as plsc

N = 16384
E = 100000
Q = 100000
REL_SRC = (0, 1, 0, 2, 1, 2)
REL_DST = (1, 0, 2, 0, 2, 1)

HALF = N // 2
NTILES = 16
EPT = 6256
EPAD = NTILES * EPT
CHUNK = 2048
CAP = 8192

_i32 = jnp.int32
_f32 = jnp.float32


ROWSZ = HALF + 128


def _bincount_body(nrows, idx_hbm, out_hbm, acc, dstbuf, didx0, didx1, ones,
                   zeros, sem0, sem1):
    c = lax.axis_index("c")
    s = lax.axis_index("s")
    didx = (didx0, didx1)
    sems = (sem0, sem1)

    def _fill(ref, n, val):
        def step(i, _):
            ref[pl.ds(i * 16, 16)] = val
            return 0
        lax.fori_loop(0, n // 16, step, 0)

    _fill(ones, CHUNK, jnp.full((16,), 1.0, _f32))
    _fill(zeros, 1024, jnp.zeros((16,), _f32))
    base = c * HALF

    def _senttail(i, _):
        dstbuf[pl.ds(EPT + i * 16, 16)] = jnp.full((16,), N, _i32)
        return 0
    lax.fori_loop(0, (CAP - EPT) // 16, _senttail, 0)

    def _zeroall(rr, _):
        pltpu.sync_copy(zeros.at[pl.ds(0, 512)],
                        acc.at[pl.ds(rr * ROWSZ + s * 512, 512)])

        @pl.when(s == 0)
        def _():
            pltpu.sync_copy(zeros.at[pl.ds(0, 16)],
                            acc.at[pl.ds(rr * ROWSZ + HALF, 16)])
        return 0

    lax.fori_loop(0, nrows, _zeroall, 0)
    plsc.subcore_barrier()

    gpc = CHUNK // 16

    def relbody(rr, _):
        abase = rr * ROWSZ
        dummy = jnp.full((16,), HALF, _i32) + lax.iota(_i32, 16)
        pltpu.sync_copy(idx_hbm.at[pl.ds(rr * EPAD + s * EPT, EPT)],
                        dstbuf.at[pl.ds(0, EPT)])
        sd = [None, None]
        for j in range(CAP // CHUNK):
            b = j % 2

            def xform(i, _2):
                g = j * gpc + i
                vd = dstbuf[pl.ds(g * 16, 16)]
                loc = vd - base
                m = (loc >= 0) & (loc < HALF)
                didx[b][pl.ds(i * 16, 16)] = abase + jnp.where(m, loc, dummy)
                return 0

            if sd[b] is not None:
                sd[b].wait()
            lax.fori_loop(0, gpc, xform, 0)
            sd[b] = pltpu.async_copy(ones, acc.at[didx[b]], sems[b], add=True)
        sd[0].wait()
        sd[1].wait()
        return 0

    lax.fori_loop(0, nrows, relbody, 0)
    plsc.subcore_barrier()

    def flush(rr, _):
        pltpu.sync_copy(acc.at[pl.ds(rr * ROWSZ + s * 512, 512)],
                        out_hbm.at[pl.ds(rr * N + base + s * 512, 512)])
        return 0

    lax.fori_loop(0, nrows, flush, 0)


def _sc_bincount(idx_pad, nrows):
    mesh = plsc.VectorSubcoreMesh(core_axis_name="c", subcore_axis_name="s")
    kern = functools.partial(
        pl.kernel,
        out_type=jax.ShapeDtypeStruct((nrows * N,), _f32),
        mesh=mesh,
        scratch_types=[
            pltpu.VMEM_SHARED((nrows * ROWSZ,), _f32),
            pltpu.VMEM((CAP,), _i32),
            pltpu.VMEM((CHUNK,), _i32),
            pltpu.VMEM((CHUNK,), _i32),
            pltpu.VMEM((CHUNK,), _f32),
            pltpu.VMEM((1024,), _f32),
            pltpu.SemaphoreType.DMA,
            pltpu.SemaphoreType.DMA,
        ],
    )(functools.partial(_bincount_body, nrows))
    return kern(idx_pad)


CHUNKR = 256
NCHR = -(-EPT // CHUNKR)
CAPR = NCHR * CHUNKR
PAD = 16


def _segsum_body(nrel, npass, ar, feats_hbm, eidx_hbm, out_hbm, acc, srcbuf,
                 dstbuf, gidx0, gidx1, didx0, didx1, rows0, rows1, zrows,
                 gsem0, gsem1, ssem0, ssem1):
    c = lax.axis_index("c")
    s = lax.axis_index("s")
    rpt = ar // 16
    gidx = (gidx0, gidx1)
    didx = (didx0, didx1)
    rows = (rows0, rows1)
    gsem = (gsem0, gsem1)
    ssem = (ssem0, ssem1)

    def _z(i, _):
        zrows[i // 8, pl.ds((i % 8) * 16, 16)] = jnp.zeros((16,), _f32)
        return 0

    lax.fori_loop(0, 32 * 8, _z, 0)

    def _tails(i, _):
        srcbuf[pl.ds(EPT + i * 16, 16)] = lax.iota(_i32, 16)
        dstbuf[pl.ds(EPT + i * 16, 16)] = jnp.full((16,), N, _i32)
        return 0

    lax.fori_loop(0, (CAPR - EPT) // 16, _tails, 0)
    gpc = CHUNKR // 16

    def relbody(rr, _):
        rbase = rr * N
        pltpu.sync_copy(eidx_hbm.at[pl.ds((2 * rr) * EPAD + s * EPT, EPT)],
                        srcbuf.at[pl.ds(0, EPT)])
        pltpu.sync_copy(eidx_hbm.at[pl.ds((2 * rr + 1) * EPAD + s * EPT, EPT)],
                        dstbuf.at[pl.ds(0, EPT)])
        for p in range(npass):
            base = (c * npass + p) * ar

            def _zero(k, _2):
                pltpu.sync_copy(zrows, acc.at[pl.ds(s * rpt + k * 32, 32)])
                return 0

            lax.fori_loop(0, rpt // 32, _zero, 0)

            @pl.when(s == 0)
            def _():
                pltpu.sync_copy(zrows.at[pl.ds(0, PAD)], acc.at[pl.ds(ar, PAD)])

            plsc.subcore_barrier()
            dummy = jnp.full((16,), ar, _i32) + lax.iota(_i32, 16)

            def xform(j, b):
                def step(i, _2):
                    g = j * gpc + i
                    vd = dstbuf[pl.ds(g * 16, 16)]
                    loc = vd - base
                    m = (loc >= 0) & (loc < ar)
                    didx[b][pl.ds(i * 16, 16)] = jnp.where(m, loc, dummy)
                    gidx[b][pl.ds(i * 16, 16)] = srcbuf[pl.ds(g * 16, 16)] + rbase
                    return 0

                lax.fori_loop(0, gpc, step, 0)

            gd = [None, None]
            sd = [None, None]
            for j in range(NCHR + 1):
                b = j % 2
                if j < NCHR:
                    if j >= 2:
                        sd[b].wait()
                    xform(j, b)
                    gd[b] = pltpu.async_copy(feats_hbm.at[gidx[b]], rows[b],
                                             gsem[b])
                if j >= 1:
                    bb = (j - 1) % 2
                    gd[bb].wait()
                    sd[bb] = pltpu.async_copy(rows[bb], acc.at[didx[bb]],
                                              ssem[bb], add=True)
            sd[(NCHR - 1) % 2].wait()
            sd[NCHR % 2].wait()
            plsc.subcore_barrier()
            pltpu.sync_copy(acc.at[pl.ds(s * rpt, rpt)],
                            out_hbm.at[pl.ds(rbase + base + s * rpt, rpt)])
            plsc.subcore_barrier()
        return 0

    lax.fori_loop(0, nrel, relbody, 0)


def _sc_segsum(feats, eidx, nrel, npass):
    ar = N // (2 * npass)
    mesh = plsc.VectorSubcoreMesh(core_axis_name="c", subcore_axis_name="s")
    kern = functools.partial(
        pl.kernel,
        out_type=jax.ShapeDtypeStruct((nrel * N, 128), _f32),
        mesh=mesh,
        scratch_types=[
            pltpu.VMEM_SHARED((ar + PAD, 128), _f32),
            pltpu.VMEM((CAPR,), _i32),
            pltpu.VMEM((CAPR,), _i32),
            pltpu.VMEM((CHUNKR,), _i32),
            pltpu.VMEM((CHUNKR,), _i32),
            pltpu.VMEM((CHUNKR,), _i32),
            pltpu.VMEM((CHUNKR,), _i32),
            pltpu.VMEM((CHUNKR, 128), _f32),
            pltpu.VMEM((CHUNKR, 128), _f32),
            pltpu.VMEM((32, 128), _f32),
            pltpu.SemaphoreType.DMA,
            pltpu.SemaphoreType.DMA,
            pltpu.SemaphoreType.DMA,
            pltpu.SemaphoreType.DMA,
        ],
    )(functools.partial(_segsum_body, nrel, npass, ar))
    return kern(feats, eidx)


QBLK = 1000


def _mlp_body(a_ref, b_ref, p2w_ref, p2b_ref, p3w_ref, p3b_ref, o_ref):
    z = jax.nn.relu(a_ref[...] + b_ref[...])
    z = jax.nn.relu(
        jax.lax.dot_general(z, p2w_ref[...], (((1,), (0,)), ((), ())))
        + p2b_ref[...])
    o_ref[...] = (
        jax.lax.dot_general(z, p3w_ref[...], (((1,), (0,)), ((), ())))
        + p3b_ref[...])


def _predictor_mlp(ag, bg, P2W, P2b, P3W, P3b):
    grid = (Q // QBLK,)
    return pl.pallas_call(
        _mlp_body,
        grid=grid,
        in_specs=[
            pl.BlockSpec((QBLK, 128), lambda i: (i, 0)),
            pl.BlockSpec((QBLK, 128), lambda i: (i, 0)),
            pl.BlockSpec((128, 64), lambda i: (0, 0)),
            pl.BlockSpec((64,), lambda i: (0,)),
            pl.BlockSpec((64, 1), lambda i: (0, 0)),
            pl.BlockSpec((1,), lambda i: (0,)),
        ],
        out_specs=pl.BlockSpec((QBLK, 1), lambda i: (i, 0)),
        out_shape=jax.ShapeDtypeStruct((Q, 1), jnp.float32),
    )(ag, bg, P2W, P2b, P3W, P3b)


def _matmul_body(x_ref, w_ref, b_ref, o_ref):
    o_ref[...] = (
        jax.lax.dot_general(x_ref[...], w_ref[...], (((1,), (0,)), ((), ())))
        + b_ref[...])


def _matmul_bias(x, w, b, blk=512):
    n, k = x.shape
    m = w.shape[1]
    return pl.pallas_call(
        _matmul_body,
        grid=(n // blk,),
        in_specs=[
            pl.BlockSpec((blk, k), lambda i: (i, 0)),
            pl.BlockSpec((k, m), lambda i: (0, 0)),
            pl.BlockSpec((m,), lambda i: (0,)),
        ],
        out_specs=pl.BlockSpec((blk, m), lambda i: (i, 0)),
        out_shape=jax.ShapeDtypeStruct((n, m), jnp.float32),
    )(x, w, b)


def kernel(x_drug, x_protein, drug_id, dis_id, edges, Wd, bd, Wp, bp, W1, b1, W2, b2, P1W, P1b, P2W, P2b, P3W, P3b):
    idx12 = edges.reshape(12, E)
    idx_pad = jnp.concatenate(
        [idx12, jnp.full((12, EPAD - E), N, _i32)], axis=1).reshape(-1)
    counts = _sc_bincount(idx_pad, 12).reshape(12, N)
    scale = jax.lax.rsqrt(jnp.maximum(counts, 1.0))

    h = {0: _matmul_bias(x_drug, Wd, bd), 1: _matmul_bias(x_protein, Wp, bp)}

    pad_src = jnp.arange(EPAD - E, dtype=_i32) % 16
    pad_dst = jnp.full((EPAD - E,), N, _i32)

    def conv(h_in, rels, W, b, npass):
        feats = jnp.concatenate(
            [_matmul_bias(h_in[REL_SRC[r]] * scale[2 * r][:, None], W[r],
                          jnp.zeros((128,), _f32)) for r in rels], axis=0)
        eidx = jnp.concatenate(
            [jnp.concatenate([edges[r, 0], pad_src, edges[r, 1], pad_dst])
             for r in rels])
        agg = _sc_segsum(feats, eidx, len(rels), npass)
        out = {}
        for k, r in enumerate(rels):
            d = REL_DST[r]
            m = (agg[k * N:(k + 1) * N] * scale[2 * r + 1][:, None] + b[r])
            out[d] = out[d] + m if d in out else m
        return out

    out1 = conv(h, (0, 1, 2, 4), W1, b1, npass=2)
    h1 = {k: jax.nn.relu(v) for k, v in out1.items()}
    out2 = conv(h1, (1, 2, 3, 4), W2, b2, npass=2)

    a = _matmul_bias(out2[0], P1W[:128], P1b)
    bz = _matmul_bias(out2[2], P1W[128:], jnp.zeros((128,), _f32))
    ag = a[drug_id]
    bg = bz[dis_id]
    return _predictor_mlp(ag, bg, P2W, P2b, P3W, P3b)
